# baseline (speedup 1.0000x reference)
import functools

import jax
import jax.numpy as jnp
from jax import lax
from jax.experimental import pallas as pl
from jax.experimental.pallas import tpu as pltpu
from jax.experimental.pallas import tpu_sc as plsc

_S = 8
_D = 1024
_H1 = 512
_H2 = 256
_N = 8192

_TILE = 256
_NT = _N // _TILE + _S
_NPAD = _NT * _TILE

_NC = 2
_NS = 16
_NW = _NC * _NS
_BPW = _NPAD // _NW
_CHUNK = 64
_NCH = _BPW // _CHUNK


def _routing(species):
    species = species.astype(jnp.int32)
    counts = jnp.bincount(species, length=_S).astype(jnp.int32)
    starts = (jnp.cumsum(counts) - counts).astype(jnp.int32)
    perm = jnp.argsort(species).astype(jnp.int32)
    ptiles = (counts + _TILE - 1) // _TILE
    pstart = (jnp.cumsum(ptiles) - ptiles).astype(jnp.int32)
    t = jnp.arange(_NT, dtype=jnp.int32)
    ts = jnp.sum((t[:, None] >= pstart[None, :]).astype(jnp.int32), axis=1) - 1
    ts = jnp.clip(ts, 0, _S - 1).astype(jnp.int32)
    i = jnp.arange(_NPAD, dtype=jnp.int32)
    tl = i // _TILE
    st = ts[tl]
    off = (tl - pstart[st]) * _TILE + (i % _TILE)
    src = starts[st] + jnp.minimum(off, jnp.maximum(counts[st] - 1, 0))
    g = perm[jnp.clip(src, 0, _N - 1)]
    return ts, pstart, counts, g


def _sc_gather(aev, idx):
    mesh = plsc.VectorSubcoreMesh(core_axis_name="c", subcore_axis_name="s")

    @functools.partial(
        pl.kernel,
        out_type=jax.ShapeDtypeStruct((_NPAD, _D), jnp.float32),
        mesh=mesh,
        scratch_types=[
            pltpu.VMEM((_NCH, _CHUNK), jnp.int32),
            pltpu.VMEM((_CHUNK, _D), jnp.float32),
            pltpu.SemaphoreType.DMA,
        ],
    )
    def gather_kernel(aev_hbm, idx_hbm, out_hbm, idx_v, rows_v, sem):
        wid = lax.axis_index("s") * _NC + lax.axis_index("c")
        pltpu.sync_copy(idx_hbm.at[wid], idx_v)
        base = wid * _BPW
        for c in range(_NCH):
            pltpu.async_copy(aev_hbm.at[idx_v.at[c]], rows_v, sem).wait()
            pltpu.sync_copy(rows_v, out_hbm.at[pl.ds(base + c * _CHUNK, _CHUNK)])

    return gather_kernel(aev, idx.reshape(_NW, _NCH, _CHUNK))


def _mlp_body(ts_ref, pstart_ref, counts_ref, b3_ref,
              x_ref, W1_ref, b1_ref, W2_ref, b2_ref, w3_ref, out_ref):
    i = pl.program_id(0)
    s = ts_ref[i]
    h = jnp.dot(x_ref[...], W1_ref[0], preferred_element_type=jnp.float32)
    h = h + b1_ref[0]
    h = jnp.where(h > 0.0, h, jnp.exp(h) - 1.0)
    h = jnp.dot(h, W2_ref[0], preferred_element_type=jnp.float32)
    h = h + b2_ref[0]
    h = jnp.where(h > 0.0, h, jnp.exp(h) - 1.0)
    row = lax.broadcasted_iota(jnp.int32, (_TILE, 1), 0)
    off0 = (i - pstart_ref[s]) * _TILE
    valid = ((row + off0) < counts_ref[s]).astype(jnp.float32)
    hsum = jnp.sum(h * valid, axis=0, keepdims=True)
    tile_e = jnp.sum(hsum * w3_ref[0]) + b3_ref[s] * jnp.sum(valid)

    @pl.when(i == 0)
    def _init():
        out_ref[0, 0] = 0.0

    out_ref[0, 0] += tile_e


def _mlp_call(xg, ts, pstart, counts, W1, b1, W2, b2, W3, b3):
    grid_spec = pltpu.PrefetchScalarGridSpec(
        num_scalar_prefetch=4,
        grid=(_NT,),
        in_specs=[
            pl.BlockSpec((_TILE, _D), lambda i, ts, ps, cn, b3: (i, 0)),
            pl.BlockSpec((1, _D, _H1), lambda i, ts, ps, cn, b3: (ts[i], 0, 0)),
            pl.BlockSpec((1, 1, _H1), lambda i, ts, ps, cn, b3: (ts[i], 0, 0)),
            pl.BlockSpec((1, _H1, _H2), lambda i, ts, ps, cn, b3: (ts[i], 0, 0)),
            pl.BlockSpec((1, 1, _H2), lambda i, ts, ps, cn, b3: (ts[i], 0, 0)),
            pl.BlockSpec((1, 1, _H2), lambda i, ts, ps, cn, b3: (ts[i], 0, 0)),
        ],
        out_specs=pl.BlockSpec(memory_space=pltpu.SMEM),
    )
    return pl.pallas_call(
        _mlp_body,
        grid_spec=grid_spec,
        out_shape=jax.ShapeDtypeStruct((1, 1), jnp.float32),
    )(ts, pstart, counts, b3.reshape(_S),
      xg, W1, b1.reshape(_S, 1, _H1), W2, b2.reshape(_S, 1, _H2),
      W3.reshape(_S, 1, _H2))


def kernel(aev, species, W1, b1, W2, b2, W3, b3):
    ts, pstart, counts, g = _routing(species)
    xg = _sc_gather(aev, g)
    total = _mlp_call(xg, ts, pstart, counts, W1, b1, W2, b2, W3, b3)
    return total.reshape(1)
